# baseline (speedup 1.0000x reference)
import functools

import jax
import jax.numpy as jnp
from jax import lax
from jax.experimental import pallas as pl
from jax.experimental.pallas import tpu as pltpu
from jax.experimental.pallas import tpu_sc as plsc

_TMP = 0.1
_FMP = 0.1
_NC = 2
_NS = 16
_RB = 8
_B = 64
_Fd = 128
_T = 4000
_TLEN = int(_T * _TMP)
_FLEN = int(_Fd * _FMP)
_CHUNKS_PER_BATCH = _Fd // _RB
_BATCH_PER_W = _B // (_NC * _NS)
_NCH = _BATCH_PER_W * _CHUNKS_PER_BATCH


def _zero16(buf, cur, j, start):
    buf[cur, j, pl.ds(start, 16)] = jnp.zeros((16,), jnp.float32)


def _mask_chunk(buf, cur, c, wid, t0_s, f0_s):
    bb = 2 * wid + lax.div(c, _CHUNKS_PER_BATCH)
    r0 = lax.rem(c, _CHUNKS_PER_BATCH) * _RB
    t0b = t0_s[pl.ds(bb, 16)][0]
    f0b = f0_s[pl.ds(bb, 16)][0]
    lo = lax.div(t0b, 16) * 16
    hi = lax.div(t0b + _TLEN, 16) * 16
    nmid = lax.div(hi - lo, 16) - 1
    col = jax.lax.broadcasted_iota(jnp.int32, (16,), 0)

    def row_body(j, _):
        v = buf[cur, j, pl.ds(lo, 16)]
        m = ((col + lo) >= t0b) & ((col + lo) < t0b + _TLEN)
        buf[cur, j, pl.ds(lo, 16)] = jnp.where(m, jnp.float32(0.0), v)

        def mid_body(k, _):
            _zero16(buf, cur, j, lo + 16 + 16 * k)
            return ()

        lax.fori_loop(0, nmid, mid_body, (), unroll=False)

        @pl.when(hi < _T)
        def _edge2():
            v2 = buf[cur, j, pl.ds(hi, 16)]
            m2 = ((col + hi) >= t0b) & ((col + hi) < t0b + _TLEN)
            buf[cur, j, pl.ds(hi, 16)] = jnp.where(m2, jnp.float32(0.0), v2)

        grow = r0 + j

        @pl.when((grow >= f0b) & (grow < f0b + _FLEN))
        def _frow():
            def z_body(k, _):
                _zero16(buf, cur, j, 16 * k)
                return ()

            lax.fori_loop(0, _T // 16, z_body, (), unroll=False)

        return ()

    lax.fori_loop(0, _RB, row_body, (), unroll=False)
    return bb, r0


def _sc_body(x_hbm, t0_hbm, f0_hbm, out_hbm, buf, t0_s, f0_s,
             insem0, insem1, outsem0, outsem1):
    wid = lax.axis_index("s") * _NC + lax.axis_index("c")
    pltpu.sync_copy(t0_hbm, t0_s)
    pltpu.sync_copy(f0_hbm, f0_s)

    def src(c):
        bb = 2 * wid + lax.div(c, _CHUNKS_PER_BATCH)
        r0 = lax.rem(c, _CHUNKS_PER_BATCH) * _RB
        return x_hbm.at[bb, pl.ds(r0, _RB)]

    def dst(c):
        bb = 2 * wid + lax.div(c, _CHUNKS_PER_BATCH)
        r0 = lax.rem(c, _CHUNKS_PER_BATCH) * _RB
        return out_hbm.at[bb, pl.ds(r0, _RB)]

    pltpu.make_async_copy(src(0), buf.at[0], insem0).start()

    def process(c, cur, insem_cur, insem_nxt, outsem_cur, outsem_nxt, nxt):
        @pl.when(c + 1 < _NCH)
        def _prefetch():
            @pl.when(c >= 1)
            def _free():
                pltpu.make_async_copy(buf.at[nxt], dst(c - 1), outsem_nxt).wait()

            pltpu.make_async_copy(src(c + 1), buf.at[nxt], insem_nxt).start()

        pltpu.make_async_copy(src(c), buf.at[cur], insem_cur).wait()
        _mask_chunk(buf, cur, c, wid, t0_s, f0_s)
        pltpu.make_async_copy(buf.at[cur], dst(c), outsem_cur).start()

    def g_body(g, _):
        c0 = 2 * g
        process(c0, 0, insem0, insem1, outsem0, outsem1, 1)
        process(c0 + 1, 1, insem1, insem0, outsem1, outsem0, 0)
        return ()

    lax.fori_loop(0, _NCH // 2, g_body, (), unroll=False)
    pltpu.make_async_copy(buf.at[0], dst(_NCH - 2), outsem0).wait()
    pltpu.make_async_copy(buf.at[1], dst(_NCH - 1), outsem1).wait()


def kernel(spec):
    B, C, Fd, T = spec.shape
    tlen = int(T * _TMP)
    flen = int(Fd * _FMP)
    t0 = jax.random.randint(
        jax.random.fold_in(jax.random.key(1), 0), (B,), 0, max(1, T - tlen + 1)
    ).astype(jnp.int32)
    f0 = jax.random.randint(
        jax.random.fold_in(jax.random.key(1), 1), (B,), 0, max(1, Fd - flen + 1)
    ).astype(jnp.int32)

    t0 = jnp.pad(t0, (0, 16))
    f0 = jnp.pad(f0, (0, 16))

    x = spec.reshape(B, Fd, T)
    mesh = plsc.VectorSubcoreMesh(core_axis_name="c", subcore_axis_name="s")
    run = functools.partial(
        pl.kernel,
        out_type=jax.ShapeDtypeStruct((B, Fd, T), spec.dtype),
        mesh=mesh,
        scratch_types=[
            pltpu.VMEM((2, _RB, T), spec.dtype),
            pltpu.VMEM((B + 16,), jnp.int32),
            pltpu.VMEM((B + 16,), jnp.int32),
            pltpu.SemaphoreType.DMA,
            pltpu.SemaphoreType.DMA,
            pltpu.SemaphoreType.DMA,
            pltpu.SemaphoreType.DMA,
        ],
    )(_sc_body)
    out = run(x, t0, f0)
    return out.reshape(B, C, Fd, T)


# speedup vs baseline: 1.0014x; 1.0014x over previous
import functools

import jax
import jax.numpy as jnp
from jax import lax
from jax.experimental import pallas as pl
from jax.experimental.pallas import tpu as pltpu
from jax.experimental.pallas import tpu_sc as plsc

_TMP = 0.1
_FMP = 0.1
_NC = 2
_NS = 16
_RB = 8
_B = 64
_Fd = 128
_T = 4000
_TLEN = int(_T * _TMP)
_FLEN = int(_Fd * _FMP)
_CHUNKS_PER_BATCH = _Fd // _RB
_BATCH_PER_W = _B // (_NC * _NS)
_NCH = _BATCH_PER_W * _CHUNKS_PER_BATCH


def _zero16(buf, cur, j, start):
    buf[cur, j, pl.ds(start, 16)] = jnp.zeros((16,), jnp.float32)


def _mask_chunk(buf, cur, c, wid, t0_s, f0_s):
    bb = 2 * wid + lax.div(c, _CHUNKS_PER_BATCH)
    r0 = lax.rem(c, _CHUNKS_PER_BATCH) * _RB
    t0b = t0_s[pl.ds(bb, 16)][0]
    f0b = f0_s[pl.ds(bb, 16)][0]
    lo = lax.div(t0b, 16) * 16
    hi = lax.div(t0b + _TLEN, 16) * 16
    nmid = lax.div(hi - lo, 16) - 1
    col = jax.lax.broadcasted_iota(jnp.int32, (16,), 0)

    def row_body(j, _):
        v = buf[cur, j, pl.ds(lo, 16)]
        m = ((col + lo) >= t0b) & ((col + lo) < t0b + _TLEN)
        buf[cur, j, pl.ds(lo, 16)] = jnp.where(m, jnp.float32(0.0), v)

        def mid_body(k, _):
            _zero16(buf, cur, j, lo + 16 + 16 * k)
            return ()

        lax.fori_loop(0, nmid, mid_body, (), unroll=False)

        @pl.when(hi < _T)
        def _edge2():
            v2 = buf[cur, j, pl.ds(hi, 16)]
            m2 = ((col + hi) >= t0b) & ((col + hi) < t0b + _TLEN)
            buf[cur, j, pl.ds(hi, 16)] = jnp.where(m2, jnp.float32(0.0), v2)

        grow = r0 + j

        @pl.when((grow >= f0b) & (grow < f0b + _FLEN))
        def _frow():
            def z_body(k, _):
                _zero16(buf, cur, j, 16 * k)
                return ()

            lax.fori_loop(0, _T // 16, z_body, (), unroll=False)

        return ()

    lax.fori_loop(0, _RB, row_body, (), unroll=False)
    return bb, r0


def _sc_body(x_hbm, t0_hbm, f0_hbm, out_hbm, buf, t0_s, f0_s,
             insem0, insem1, outsem0, outsem1):
    wid = lax.axis_index("s") * _NC + lax.axis_index("c")
    pltpu.sync_copy(t0_hbm, t0_s)
    pltpu.sync_copy(f0_hbm, f0_s)

    def src(c):
        bb = 2 * wid + lax.div(c, _CHUNKS_PER_BATCH)
        r0 = lax.rem(c, _CHUNKS_PER_BATCH) * _RB
        return x_hbm.at[bb, pl.ds(r0, _RB)]

    def dst(c):
        bb = 2 * wid + lax.div(c, _CHUNKS_PER_BATCH)
        r0 = lax.rem(c, _CHUNKS_PER_BATCH) * _RB
        return out_hbm.at[bb, pl.ds(r0, _RB)]

    pltpu.make_async_copy(src(0), buf.at[0], insem0).start()

    def process(c, cur, insem_cur, insem_nxt, outsem_cur, outsem_nxt, nxt):
        @pl.when(c + 1 < _NCH)
        def _prefetch():
            @pl.when(c >= 1)
            def _free():
                pltpu.make_async_copy(buf.at[nxt], dst(c - 1), outsem_nxt).wait()

            pltpu.make_async_copy(src(c + 1), buf.at[nxt], insem_nxt).start()

        pltpu.make_async_copy(src(c), buf.at[cur], insem_cur).wait()
        _mask_chunk(buf, cur, c, wid, t0_s, f0_s)
        pltpu.make_async_copy(buf.at[cur], dst(c), outsem_cur).start()

    def g_body(g, _):
        c0 = 2 * g
        process(c0, 0, insem0, insem1, outsem0, outsem1, 1)
        process(c0 + 1, 1, insem1, insem0, outsem1, outsem0, 0)
        return ()

    lax.fori_loop(0, _NCH // 2, g_body, (), unroll=False)
    pltpu.make_async_copy(buf.at[0], dst(_NCH - 2), outsem0).wait()
    pltpu.make_async_copy(buf.at[1], dst(_NCH - 1), outsem1).wait()


def kernel(spec):
    B, C, Fd, T = spec.shape
    tlen = int(T * _TMP)
    flen = int(Fd * _FMP)
    t0 = jax.random.randint(
        jax.random.fold_in(jax.random.key(1), 0), (B,), 0, max(1, T - tlen + 1)
    ).astype(jnp.int32)
    f0 = jax.random.randint(
        jax.random.fold_in(jax.random.key(1), 1), (B,), 0, max(1, Fd - flen + 1)
    ).astype(jnp.int32)

    t0 = jnp.pad(t0, (0, 16))
    f0 = jnp.pad(f0, (0, 16))

    x = spec.reshape(B, Fd, T)
    mesh = plsc.VectorSubcoreMesh(core_axis_name="c", subcore_axis_name="s")
    run = functools.partial(
        pl.kernel,
        out_type=jax.ShapeDtypeStruct((B, Fd, T), spec.dtype),
        mesh=mesh,
        compiler_params=pltpu.CompilerParams(use_tc_tiling_on_sc=True),
        scratch_types=[
            pltpu.VMEM((2, _RB, T), spec.dtype),
            pltpu.VMEM((B + 16,), jnp.int32),
            pltpu.VMEM((B + 16,), jnp.int32),
            pltpu.SemaphoreType.DMA,
            pltpu.SemaphoreType.DMA,
            pltpu.SemaphoreType.DMA,
            pltpu.SemaphoreType.DMA,
        ],
    )(_sc_body)
    out = run(x, t0, f0)
    return out.reshape(B, C, Fd, T)
